# baseline (speedup 1.0000x reference)
import functools

import jax
import jax.numpy as jnp
from jax import lax
from jax.experimental import pallas as pl
from jax.experimental.pallas import tpu as pltpu
from jax.experimental.pallas import tpu_sc as plsc

_ALPHA = 4


@functools.lru_cache(maxsize=None)
def _make_sc_gather(C, T, H, W):
    S = T // _ALPHA
    F = H * W
    info = plsc.get_sparse_core_info()
    NW = info.num_cores * info.num_subcores
    NFR = C * S
    CHN = 1
    while (NFR * CHN) % NW != 0 or F // CHN > 49152:
        CHN += 1
    PIECE = F // CHN
    PPW = (NFR * CHN) // NW

    mesh = plsc.VectorSubcoreMesh(core_axis_name="c", subcore_axis_name="s")

    @functools.partial(
        pl.kernel,
        mesh=mesh,
        out_type=jax.ShapeDtypeStruct((C * S * F,), jnp.float32),
        scratch_types=[
            pltpu.VMEM((PIECE,), jnp.float32),
            pltpu.VMEM((PIECE,), jnp.float32),
            pltpu.SemaphoreType.DMA,
            pltpu.SemaphoreType.DMA,
            pltpu.SemaphoreType.DMA,
            pltpu.SemaphoreType.DMA,
        ],
    )
    def gather(frames_hbm, out_hbm, buf0, buf1, isem0, isem1, osem0, osem1):
        wid = lax.axis_index("s") * info.num_cores + lax.axis_index("c")
        bufs = (buf0, buf1)
        isems = (isem0, isem1)
        osems = (osem0, osem1)

        def offs(p):
            pid = wid * PPW + p
            c = pid // (S * CHN)
            rem = pid % (S * CHN)
            j = rem // CHN
            k = rem % CHN
            t = (j * (T - 1)) // (S - 1)
            src = pl.multiple_of((c * T + t) * F + k * PIECE, 8)
            dst = pl.multiple_of(pid * PIECE, 8)
            return src, dst

        in_cp = [None] * PPW
        out_cp = [None] * PPW
        for p in range(PPW):
            s = p % 2
            src, _ = offs(p)
            if p >= 2:
                out_cp[p - 2].wait()
            in_cp[p] = pltpu.make_async_copy(
                frames_hbm.at[pl.ds(src, PIECE)], bufs[s], isems[s])
            in_cp[p].start()
            if p >= 1:
                _, dst = offs(p - 1)
                in_cp[p - 1].wait()
                out_cp[p - 1] = pltpu.make_async_copy(
                    bufs[(p - 1) % 2], out_hbm.at[pl.ds(dst, PIECE)],
                    osems[(p - 1) % 2])
                out_cp[p - 1].start()
        _, dst = offs(PPW - 1)
        in_cp[PPW - 1].wait()
        out_cp[PPW - 1] = pltpu.make_async_copy(
            bufs[(PPW - 1) % 2], out_hbm.at[pl.ds(dst, PIECE)],
            osems[(PPW - 1) % 2])
        out_cp[PPW - 1].start()
        if PPW >= 2:
            out_cp[PPW - 2].wait()
        out_cp[PPW - 1].wait()

    return gather


def kernel(frames):
    C, T, H, W = frames.shape
    slow_flat = _make_sc_gather(C, T, H, W)(frames.reshape(-1))
    slow = slow_flat.reshape(C, T // _ALPHA, H, W)
    return (slow, frames)


# speedup vs baseline: 2.0432x; 2.0432x over previous
import functools

import jax
import jax.numpy as jnp
from jax import lax
from jax.experimental import pallas as pl
from jax.experimental.pallas import tpu as pltpu
from jax.experimental.pallas import tpu_sc as plsc

_ALPHA = 4


@functools.lru_cache(maxsize=None)
def _make_sc_gather(C, T, H, W):
    S = T // _ALPHA
    info = plsc.get_sparse_core_info()
    NW = info.num_cores * info.num_subcores
    NFR = C * S
    CHN = 1
    while ((NFR * CHN) % NW != 0 or (H // CHN) * W > 49152
           or H % CHN != 0 or (H // CHN) % 8 != 0):
        CHN += 1
    ROWS = H // CHN
    PPW = (NFR * CHN) // NW

    mesh = plsc.VectorSubcoreMesh(core_axis_name="c", subcore_axis_name="s")

    @functools.partial(
        pl.kernel,
        mesh=mesh,
        out_type=jax.ShapeDtypeStruct((C, S, H, W), jnp.float32),
        scratch_types=[
            pltpu.VMEM((ROWS, W), jnp.float32),
            pltpu.VMEM((ROWS, W), jnp.float32),
            pltpu.SemaphoreType.DMA,
            pltpu.SemaphoreType.DMA,
            pltpu.SemaphoreType.DMA,
            pltpu.SemaphoreType.DMA,
        ],
    )
    def gather(frames_hbm, out_hbm, buf0, buf1, isem0, isem1, osem0, osem1):
        wid = lax.axis_index("s") * info.num_cores + lax.axis_index("c")
        bufs = (buf0, buf1)
        isems = (isem0, isem1)
        osems = (osem0, osem1)

        def coords(p):
            pid = wid * PPW + p
            c = pid // (S * CHN)
            rem = pid % (S * CHN)
            j = rem // CHN
            k = rem % CHN
            t = (j * (T - 1)) // (S - 1)
            return c, t, j, k

        in_cp = [None] * PPW
        out_cp = [None] * PPW
        for p in range(PPW):
            s = p % 2
            c, t, _, k = coords(p)
            if p >= 2:
                out_cp[p - 2].wait()
            in_cp[p] = pltpu.make_async_copy(
                frames_hbm.at[c, t, pl.ds(k * ROWS, ROWS), :],
                bufs[s], isems[s])
            in_cp[p].start()
            if p >= 1:
                c, _, j, k = coords(p - 1)
                in_cp[p - 1].wait()
                out_cp[p - 1] = pltpu.make_async_copy(
                    bufs[(p - 1) % 2],
                    out_hbm.at[c, j, pl.ds(k * ROWS, ROWS), :],
                    osems[(p - 1) % 2])
                out_cp[p - 1].start()
        c, _, j, k = coords(PPW - 1)
        in_cp[PPW - 1].wait()
        out_cp[PPW - 1] = pltpu.make_async_copy(
            bufs[(PPW - 1) % 2],
            out_hbm.at[c, j, pl.ds(k * ROWS, ROWS), :],
            osems[(PPW - 1) % 2])
        out_cp[PPW - 1].start()
        if PPW >= 2:
            out_cp[PPW - 2].wait()
        out_cp[PPW - 1].wait()

    return gather


def kernel(frames):
    C, T, H, W = frames.shape
    slow = _make_sc_gather(C, T, H, W)(frames)
    return (slow, frames)


# speedup vs baseline: 2.0989x; 1.0273x over previous
import functools

import jax
import jax.numpy as jnp
from jax import lax
from jax.experimental import pallas as pl
from jax.experimental.pallas import tpu as pltpu
from jax.experimental.pallas import tpu_sc as plsc

_ALPHA = 4


@functools.lru_cache(maxsize=None)
def _make_sc_gather(C, T, H, W):
    S = T // _ALPHA
    info = plsc.get_sparse_core_info()
    NW = info.num_cores * info.num_subcores
    NFR = C * S
    CHN = 1
    while ((NFR * CHN) % NW != 0 or (H // CHN) * W > 49152
           or H % CHN != 0 or (H // CHN) % 8 != 0):
        CHN += 1
    ROWS = H // CHN
    PPW = (NFR * CHN) // NW

    mesh = plsc.VectorSubcoreMesh(core_axis_name="c", subcore_axis_name="s")

    @functools.partial(
        pl.kernel,
        mesh=mesh,
        out_type=jax.ShapeDtypeStruct((C, S, H, W), jnp.float32),
        scratch_types=[
            pltpu.VMEM((ROWS, W), jnp.float32),
            pltpu.VMEM((ROWS, W), jnp.float32),
            pltpu.SemaphoreType.DMA,
            pltpu.SemaphoreType.DMA,
            pltpu.SemaphoreType.DMA,
            pltpu.SemaphoreType.DMA,
        ],
    )
    def gather(frames_hbm, out_hbm, buf0, buf1, isem0, isem1, osem0, osem1):
        wid = lax.axis_index("s") * info.num_cores + lax.axis_index("c")
        bufs = (buf0, buf1)
        isems = (isem0, isem1)
        osems = (osem0, osem1)

        def coords(p):
            pid = wid * PPW + p
            c = pid // (S * CHN)
            rem = pid % (S * CHN)
            j = rem // CHN
            k = rem % CHN
            t = (j * (T - 1)) // (S - 1)
            return c, t, j, k

        in_cp = [None] * PPW
        out_cp = [None] * PPW
        for p in range(PPW):
            s = p % 2
            c, t, _, k = coords(p)
            if p >= 2:
                out_cp[p - 2].wait()
            in_cp[p] = pltpu.make_async_copy(
                frames_hbm.at[c, t, pl.ds(k * ROWS, ROWS), :],
                bufs[s], isems[s])
            in_cp[p].start()
            if p >= 1:
                c, _, j, k = coords(p - 1)
                in_cp[p - 1].wait()
                out_cp[p - 1] = pltpu.make_async_copy(
                    bufs[(p - 1) % 2],
                    out_hbm.at[c, j, pl.ds(k * ROWS, ROWS), :],
                    osems[(p - 1) % 2])
                out_cp[p - 1].start()
        c, _, j, k = coords(PPW - 1)
        in_cp[PPW - 1].wait()
        out_cp[PPW - 1] = pltpu.make_async_copy(
            bufs[(PPW - 1) % 2],
            out_hbm.at[c, j, pl.ds(k * ROWS, ROWS), :],
            osems[(PPW - 1) % 2])
        out_cp[PPW - 1].start()
        if PPW >= 2:
            out_cp[PPW - 2].wait()
        out_cp[PPW - 1].wait()

    return gather


@functools.lru_cache(maxsize=None)
def _make_tc_copy(C, T, H, W):
    BT = 4

    def body(i_ref, o_ref):
        o_ref[...] = i_ref[...]

    return pl.pallas_call(
        body,
        grid=(C, T // BT),
        in_specs=[pl.BlockSpec((1, BT, H, W), lambda c, t: (c, t, 0, 0))],
        out_specs=pl.BlockSpec((1, BT, H, W), lambda c, t: (c, t, 0, 0)),
        out_shape=jax.ShapeDtypeStruct((C, T, H, W), jnp.float32),
    )


def kernel(frames):
    C, T, H, W = frames.shape
    slow = _make_sc_gather(C, T, H, W)(frames)
    fast = _make_tc_copy(C, T, H, W)(frames)
    return (slow, fast)
